# baseline (speedup 1.0000x reference)
import functools
import math

import jax
import jax.numpy as jnp
from jax import lax
from jax.experimental import pallas as pl
from jax.experimental.pallas import tpu as pltpu
from jax.experimental.pallas import tpu_sc as plsc

NC = 2
NS = 16
NW = NC * NS
CH = 80
NR = 4
GB = 5
BLK = 1024


def _hist_call(npad, ep, rpt):
    mesh = plsc.VectorSubcoreMesh(core_axis_name="c", subcore_axis_name="s")
    c2 = 128
    ncks = ep // c2

    @functools.partial(
        pl.kernel,
        mesh=mesh,
        out_type=jax.ShapeDtypeStruct((NC * npad,), jnp.float32),
        scratch_types=[
            pltpu.VMEM((4, 2, c2), jnp.int32),
            pltpu.VMEM((c2,), jnp.float32),
            pltpu.VMEM((rpt,), jnp.float32),
            pltpu.VMEM_SHARED((npad,), jnp.float32),
            [pltpu.SemaphoreType.DMA] * 4,
        ],
    )
    def hist(ei_hbm, ones_hbm, zeros_hbm, out_hbm,
             ibuf, ones_v, buf_v, acc_sh, sems):
        cid = lax.axis_index("c")
        sid = lax.axis_index("s")
        wid = sid * NC + cid
        pltpu.sync_copy(ones_hbm, ones_v)
        pltpu.sync_copy(zeros_hbm, buf_v)
        pltpu.sync_copy(buf_v, acc_sh.at[pl.ds(sid * rpt, rpt)])
        plsc.subcore_barrier()

        for j in range(2):
            pltpu.sync_copy(
                ei_hbm.at[:, pl.ds((wid + NW * j) * c2, c2)], ibuf.at[j])

        def group(g, carry):
            for b in range(4):
                i = g * 4 + b
                bf = (b + 2) % 4

                @pl.when(wid + NW * (i + 2) < ncks)
                def _fetch():
                    off = (wid + NW * (i + 2)) * c2
                    pltpu.async_copy(
                        ei_hbm.at[:, pl.ds(off, c2)], ibuf.at[bf], sems[bf])

                @pl.when(wid + NW * i < ncks)
                def _scat():
                    if b in (0, 1):
                        @pl.when(g > 0)
                        def _wait():
                            pltpu.make_async_copy(
                                ei_hbm.at[:, pl.ds(0, c2)], ibuf.at[b],
                                sems[b]).wait()
                    else:
                        pltpu.make_async_copy(
                            ei_hbm.at[:, pl.ds(0, c2)], ibuf.at[b],
                            sems[b]).wait()
                    pltpu.sync_copy(ones_v, acc_sh.at[ibuf.at[b, 0]], add=True)
            return carry

        nv = -(-ncks // NW)
        lax.fori_loop(0, (nv + 3) // 4, group, 0)
        plsc.subcore_barrier()
        pltpu.sync_copy(acc_sh.at[pl.ds(sid * rpt, rpt)], buf_v)
        pltpu.sync_copy(buf_v, out_hbm.at[pl.ds(cid * npad + sid * rpt, rpt)])

    return hist


def _agg_call(npad, d, ep, rpt, nch):
    mesh = plsc.VectorSubcoreMesh(core_axis_name="c", subcore_axis_name="s")
    rc = math.gcd(rpt, CH)
    NI = 6

    @functools.partial(
        pl.kernel,
        mesh=mesh,
        out_type=jax.ShapeDtypeStruct((NC * npad, d), jnp.float32),
        scratch_types=[
            pltpu.VMEM((NI, CH), jnp.int32),
            pltpu.VMEM((NI, CH), jnp.int32),
            pltpu.VMEM((NR, CH, d), jnp.float32),
            pltpu.VMEM_SHARED((npad, d), jnp.float32),
            [pltpu.SemaphoreType.DMA] * NI,
            [pltpu.SemaphoreType.DMA] * NR,
            [pltpu.SemaphoreType.DMA] * NR,
        ],
    )
    def agg(ef_hbm, y_hbm, zeros_hbm, out_hbm,
            cbuf, rbuf, rows, acc_sh, sis, sgs, sss):
        cid = lax.axis_index("c")
        sid = lax.axis_index("s")
        wid = sid * NC + cid
        pltpu.sync_copy(zeros_hbm, rows.at[0, pl.ds(0, rc)])
        pltpu.sync_copy(zeros_hbm, rows.at[1, pl.ds(0, rc)])
        for k in range(rpt // rc):
            pltpu.sync_copy(
                rows.at[k % 2, pl.ds(0, rc)],
                acc_sh.at[pl.ds(sid * rpt + k * rc, rc)])
        plsc.subcore_barrier()

        base = wid * (ep // NW)
        cbase = ep + base

        for j in range(2):
            pltpu.sync_copy(ef_hbm.at[pl.ds(cbase + j * CH, CH)], cbuf.at[j])
            pltpu.sync_copy(ef_hbm.at[pl.ds(base + j * CH, CH)], rbuf.at[j])
        pltpu.async_copy(y_hbm.at[cbuf.at[0]], rows.at[0], sgs[0])

        NV = NI * NR // 2

        def group(g, carry):
            for b in range(NV):
                i = g * NV + b
                e2 = (b + 2) % NI
                e1 = (b + 1) % NI
                r1 = (b + 1) % NR
                r0 = b % NR

                @pl.when(i + 2 < nch)
                def _fetch():
                    off = base + (i + 2) * CH
                    pltpu.async_copy(
                        ef_hbm.at[pl.ds(ep + off, CH)], cbuf.at[e2], sis[e2])
                    pltpu.async_copy(
                        ef_hbm.at[pl.ds(off, CH)], rbuf.at[e2], sis[e2])

                @pl.when(i + 1 < nch)
                def _gather():
                    if b == 0:
                        @pl.when(g > 0)
                        def _wi():
                            pltpu.make_async_copy(
                                ef_hbm.at[pl.ds(cbase, CH)], cbuf.at[e1],
                                sis[e1]).wait()
                            pltpu.make_async_copy(
                                ef_hbm.at[pl.ds(base, CH)], rbuf.at[e1],
                                sis[e1]).wait()
                    else:
                        pltpu.make_async_copy(
                            ef_hbm.at[pl.ds(cbase, CH)], cbuf.at[e1],
                            sis[e1]).wait()
                        pltpu.make_async_copy(
                            ef_hbm.at[pl.ds(base, CH)], rbuf.at[e1],
                            sis[e1]).wait()
                    if b in (0, 1, 2):
                        @pl.when(g > 0)
                        def _ws():
                            pltpu.make_async_copy(
                                rows.at[r1],
                                acc_sh.at[rbuf.at[(b + 1 - NR) % NI]],
                                sss[r1]).wait()
                    else:
                        pltpu.make_async_copy(
                            rows.at[r1],
                            acc_sh.at[rbuf.at[(b + 1 - NR) % NI]],
                            sss[r1]).wait()
                    pltpu.async_copy(
                        y_hbm.at[cbuf.at[e1]], rows.at[r1], sgs[r1])

                @pl.when(i < nch)
                def _scat():
                    pltpu.make_async_copy(
                        y_hbm.at[cbuf.at[b % NI]], rows.at[r0], sgs[r0]).wait()
                    pltpu.async_copy(
                        rows.at[r0], acc_sh.at[rbuf.at[b % NI]], sss[r0],
                        add=True)
            return carry

        lax.fori_loop(0, (nch + NV - 1) // NV, group, 0)
        for j in range(NR):
            i = nch - NR + j
            pltpu.make_async_copy(
                rows.at[i % NR], acc_sh.at[rbuf.at[i % NI]],
                sss[i % NR]).wait()
        plsc.subcore_barrier()
        for k in range(rpt // rc):
            kb = k % 2
            pltpu.sync_copy(
                acc_sh.at[pl.ds(sid * rpt + k * rc, rc)],
                rows.at[kb, pl.ds(0, rc)])
            pltpu.sync_copy(
                rows.at[kb, pl.ds(0, rc)],
                out_hbm.at[pl.ds(cid * npad + sid * rpt + k * rc, rc)])

    return agg


def _transform_kernel(d0_ref, d1_ref, x_ref, w_ref, y_ref):
    deg = d0_ref[...] + d1_ref[...] + 2.0
    dinv = lax.rsqrt(deg)
    xw = jnp.dot(x_ref[...], w_ref[...], preferred_element_type=jnp.float32)
    y_ref[...] = dinv[:, None] * xw


def _final_kernel(d0_ref, d1_ref, a0_ref, a1_ref, y_ref, o_ref):
    deg = d0_ref[...] + d1_ref[...] + 2.0
    dinv = lax.rsqrt(deg)
    s = a0_ref[...] + a1_ref[...] + 2.0 * y_ref[...]
    o_ref[...] = jnp.maximum(dinv[:, None] * s, 0.0)


def kernel(x, edge_index, W):
    n, d_in = x.shape
    d_out = W.shape[1]
    e = edge_index.shape[1]

    npad = -(-n // (NS * CH)) * (NS * CH)
    ep = -(-e // (NW * CH)) * (NW * CH)
    rpt = npad // NS
    nch = ep // (NW * CH)

    ei = edge_index.astype(jnp.int32)
    if ep != e:
        pad_idx = jnp.full((2, ep - e), npad - 1, dtype=jnp.int32)
        ei = jnp.concatenate([ei, pad_idx], axis=1)
    eflat = ei.reshape(-1)

    rc = math.gcd(rpt, CH)
    ones_ch = jnp.ones((128,), jnp.float32)
    zeros_rpt = jnp.zeros((rpt,), jnp.float32)
    zeros_blk = jnp.zeros((rc, d_out), jnp.float32)

    degp = _hist_call(npad, ep, rpt)(ei, ones_ch, zeros_rpt)

    nb = -(-n // BLK)
    nshift = npad // BLK
    y = pl.pallas_call(
        _transform_kernel,
        grid=(nb,),
        in_specs=[
            pl.BlockSpec((BLK,), lambda i: (i,)),
            pl.BlockSpec((BLK,), lambda i: (i + nshift,)),
            pl.BlockSpec((BLK, d_in), lambda i: (i, 0)),
            pl.BlockSpec((d_in, d_out), lambda i: (0, 0)),
        ],
        out_specs=pl.BlockSpec((BLK, d_out), lambda i: (i, 0)),
        out_shape=jax.ShapeDtypeStruct((n, d_out), jnp.float32),
    )(degp, degp, x, W)

    aggp = _agg_call(npad, d_out, ep, rpt, nch)(eflat, y, zeros_blk)

    out = pl.pallas_call(
        _final_kernel,
        grid=(nb,),
        in_specs=[
            pl.BlockSpec((BLK,), lambda i: (i,)),
            pl.BlockSpec((BLK,), lambda i: (i + nshift,)),
            pl.BlockSpec((BLK, d_out), lambda i: (i, 0)),
            pl.BlockSpec((BLK, d_out), lambda i: (i + nshift, 0)),
            pl.BlockSpec((BLK, d_out), lambda i: (i, 0)),
        ],
        out_specs=pl.BlockSpec((BLK, d_out), lambda i: (i, 0)),
        out_shape=jax.ShapeDtypeStruct((n, d_out), jnp.float32),
    )(degp, degp, aggp, aggp, y)

    return out


# speedup vs baseline: 1.0410x; 1.0410x over previous
import functools
import math

import jax
import jax.numpy as jnp
from jax import lax
from jax.experimental import pallas as pl
from jax.experimental.pallas import tpu as pltpu
from jax.experimental.pallas import tpu_sc as plsc

NC = 2
NS = 16
NW = NC * NS
CH = 80
NR = 4
GB = 5
BLK = 2048


def _hist_call(npad, ep, rpt):
    mesh = plsc.VectorSubcoreMesh(core_axis_name="c", subcore_axis_name="s")
    c2 = 128
    ncks = ep // c2

    @functools.partial(
        pl.kernel,
        mesh=mesh,
        out_type=jax.ShapeDtypeStruct((NC * npad,), jnp.float32),
        scratch_types=[
            pltpu.VMEM((4, 2, c2), jnp.int32),
            pltpu.VMEM((c2,), jnp.float32),
            pltpu.VMEM((rpt,), jnp.float32),
            pltpu.VMEM_SHARED((npad,), jnp.float32),
            [pltpu.SemaphoreType.DMA] * 4,
        ],
    )
    def hist(ei_hbm, ones_hbm, zeros_hbm, out_hbm,
             ibuf, ones_v, buf_v, acc_sh, sems):
        cid = lax.axis_index("c")
        sid = lax.axis_index("s")
        wid = sid * NC + cid
        pltpu.sync_copy(ones_hbm, ones_v)
        pltpu.sync_copy(zeros_hbm, buf_v)
        pltpu.sync_copy(buf_v, acc_sh.at[pl.ds(sid * rpt, rpt)])
        plsc.subcore_barrier()

        for j in range(2):
            pltpu.sync_copy(
                ei_hbm.at[:, pl.ds((wid + NW * j) * c2, c2)], ibuf.at[j])

        def group(g, carry):
            for b in range(4):
                i = g * 4 + b
                bf = (b + 2) % 4

                @pl.when(wid + NW * (i + 2) < ncks)
                def _fetch():
                    off = (wid + NW * (i + 2)) * c2
                    pltpu.async_copy(
                        ei_hbm.at[:, pl.ds(off, c2)], ibuf.at[bf], sems[bf])

                @pl.when(wid + NW * i < ncks)
                def _scat():
                    if b in (0, 1):
                        @pl.when(g > 0)
                        def _wait():
                            pltpu.make_async_copy(
                                ei_hbm.at[:, pl.ds(0, c2)], ibuf.at[b],
                                sems[b]).wait()
                    else:
                        pltpu.make_async_copy(
                            ei_hbm.at[:, pl.ds(0, c2)], ibuf.at[b],
                            sems[b]).wait()
                    pltpu.sync_copy(ones_v, acc_sh.at[ibuf.at[b, 0]], add=True)
            return carry

        nv = -(-ncks // NW)
        lax.fori_loop(0, (nv + 3) // 4, group, 0)
        plsc.subcore_barrier()
        pltpu.sync_copy(acc_sh.at[pl.ds(sid * rpt, rpt)], buf_v)
        pltpu.sync_copy(buf_v, out_hbm.at[pl.ds(cid * npad + sid * rpt, rpt)])

    return hist


def _agg_call(npad, d, ep, rpt, nch):
    mesh = plsc.VectorSubcoreMesh(core_axis_name="c", subcore_axis_name="s")
    rc = math.gcd(rpt, CH)

    @functools.partial(
        pl.kernel,
        mesh=mesh,
        out_type=jax.ShapeDtypeStruct((NC * npad, d), jnp.float32),
        scratch_types=[
            pltpu.VMEM((NR, CH), jnp.int32),
            pltpu.VMEM((NR, CH), jnp.int32),
            pltpu.VMEM((NR, CH, d), jnp.float32),
            pltpu.VMEM_SHARED((npad, d), jnp.float32),
            [pltpu.SemaphoreType.DMA] * NR,
            [pltpu.SemaphoreType.DMA] * NR,
            [pltpu.SemaphoreType.DMA] * NR,
        ],
    )
    def agg(ef_hbm, y_hbm, zeros_hbm, out_hbm,
            cbuf, rbuf, rows, acc_sh, sis, sgs, sss):
        cid = lax.axis_index("c")
        sid = lax.axis_index("s")
        wid = sid * NC + cid
        pltpu.sync_copy(zeros_hbm, rows.at[0, pl.ds(0, rc)])
        for k in range(rpt // rc):
            pltpu.sync_copy(
                rows.at[0, pl.ds(0, rc)],
                acc_sh.at[pl.ds(sid * rpt + k * rc, rc)])
        plsc.subcore_barrier()

        base = wid * (ep // NW)
        cbase = ep + base

        for j in range(2):
            pltpu.sync_copy(ef_hbm.at[pl.ds(cbase + j * CH, CH)], cbuf.at[j])
            pltpu.sync_copy(ef_hbm.at[pl.ds(base + j * CH, CH)], rbuf.at[j])
        pltpu.async_copy(y_hbm.at[cbuf.at[0]], rows.at[0], sgs[0])

        def group(g, carry):
            for b in range(NR):
                i = g * NR + b
                b2 = (b + 2) % NR
                b1 = (b + 1) % NR

                @pl.when(i + 2 < nch)
                def _fetch():
                    if b in (0, 1):
                        @pl.when(g > 0)
                        def _ws():
                            pltpu.make_async_copy(
                                rows.at[b2], acc_sh.at[rbuf.at[b2]],
                                sss[b2]).wait()
                    else:
                        pltpu.make_async_copy(
                            rows.at[b2], acc_sh.at[rbuf.at[b2]],
                            sss[b2]).wait()
                    off = base + (i + 2) * CH
                    pltpu.async_copy(
                        ef_hbm.at[pl.ds(ep + off, CH)], cbuf.at[b2], sis[b2])
                    pltpu.async_copy(
                        ef_hbm.at[pl.ds(off, CH)], rbuf.at[b2], sis[b2])

                @pl.when(i + 1 < nch)
                def _gather():
                    if b == 0:
                        @pl.when(g > 0)
                        def _wi():
                            pltpu.make_async_copy(
                                ef_hbm.at[pl.ds(cbase, CH)], cbuf.at[b1],
                                sis[b1]).wait()
                            pltpu.make_async_copy(
                                ef_hbm.at[pl.ds(base, CH)], rbuf.at[b1],
                                sis[b1]).wait()
                    else:
                        pltpu.make_async_copy(
                            ef_hbm.at[pl.ds(cbase, CH)], cbuf.at[b1],
                            sis[b1]).wait()
                        pltpu.make_async_copy(
                            ef_hbm.at[pl.ds(base, CH)], rbuf.at[b1],
                            sis[b1]).wait()
                    pltpu.async_copy(
                        y_hbm.at[cbuf.at[b1]], rows.at[b1], sgs[b1])

                @pl.when(i < nch)
                def _scat():
                    pltpu.make_async_copy(
                        y_hbm.at[cbuf.at[b]], rows.at[b], sgs[b]).wait()
                    pltpu.async_copy(
                        rows.at[b], acc_sh.at[rbuf.at[b]], sss[b], add=True)
            return carry

        lax.fori_loop(0, (nch + NR - 1) // NR, group, 0)
        for b in range(NR):
            pltpu.make_async_copy(
                rows.at[b], acc_sh.at[rbuf.at[b]], sss[b]).wait()
        plsc.subcore_barrier()
        for k in range(rpt // rc):
            pltpu.sync_copy(
                acc_sh.at[pl.ds(sid * rpt + k * rc, rc)],
                rows.at[0, pl.ds(0, rc)])
            pltpu.sync_copy(
                rows.at[0, pl.ds(0, rc)],
                out_hbm.at[pl.ds(cid * npad + sid * rpt + k * rc, rc)])

    return agg


def _transform_kernel(d0_ref, d1_ref, x_ref, w_ref, y_ref):
    deg = d0_ref[...] + d1_ref[...] + 2.0
    dinv = lax.rsqrt(deg)
    xw = jnp.dot(x_ref[...], w_ref[...], preferred_element_type=jnp.float32)
    y_ref[...] = dinv[:, None] * xw


def _final_kernel(d0_ref, d1_ref, a0_ref, a1_ref, y_ref, o_ref):
    deg = d0_ref[...] + d1_ref[...] + 2.0
    dinv = lax.rsqrt(deg)
    s = a0_ref[...] + a1_ref[...] + 2.0 * y_ref[...]
    o_ref[...] = jnp.maximum(dinv[:, None] * s, 0.0)


def kernel(x, edge_index, W):
    n, d_in = x.shape
    d_out = W.shape[1]
    e = edge_index.shape[1]

    npad = -(-n // (NS * CH)) * (NS * CH)
    ep = -(-e // (NW * CH)) * (NW * CH)
    rpt = npad // NS
    nch = ep // (NW * CH)

    ei = edge_index.astype(jnp.int32)
    if ep != e:
        pad_idx = jnp.full((2, ep - e), npad - 1, dtype=jnp.int32)
        ei = jnp.concatenate([ei, pad_idx], axis=1)
    eflat = ei.reshape(-1)

    rc = math.gcd(rpt, CH)
    ones_ch = jnp.ones((128,), jnp.float32)
    zeros_rpt = jnp.zeros((rpt,), jnp.float32)
    zeros_blk = jnp.zeros((rc, d_out), jnp.float32)

    degp = _hist_call(npad, ep, rpt)(ei, ones_ch, zeros_rpt)

    nb = -(-n // BLK)
    nshift = npad // BLK
    y = pl.pallas_call(
        _transform_kernel,
        grid=(nb,),
        in_specs=[
            pl.BlockSpec((BLK,), lambda i: (i,)),
            pl.BlockSpec((BLK,), lambda i: (i + nshift,)),
            pl.BlockSpec((BLK, d_in), lambda i: (i, 0)),
            pl.BlockSpec((d_in, d_out), lambda i: (0, 0)),
        ],
        out_specs=pl.BlockSpec((BLK, d_out), lambda i: (i, 0)),
        out_shape=jax.ShapeDtypeStruct((n, d_out), jnp.float32),
    )(degp, degp, x, W)

    aggp = _agg_call(npad, d_out, ep, rpt, nch)(eflat, y, zeros_blk)

    out = pl.pallas_call(
        _final_kernel,
        grid=(nb,),
        in_specs=[
            pl.BlockSpec((BLK,), lambda i: (i,)),
            pl.BlockSpec((BLK,), lambda i: (i + nshift,)),
            pl.BlockSpec((BLK, d_out), lambda i: (i, 0)),
            pl.BlockSpec((BLK, d_out), lambda i: (i + nshift, 0)),
            pl.BlockSpec((BLK, d_out), lambda i: (i, 0)),
        ],
        out_specs=pl.BlockSpec((BLK, d_out), lambda i: (i, 0)),
        out_shape=jax.ShapeDtypeStruct((n, d_out), jnp.float32),
    )(degp, degp, aggp, aggp, y)

    return out
